# baseline (speedup 1.0000x reference)
import functools

import jax
import jax.numpy as jnp
from jax.experimental import pallas as pl
from jax.experimental.pallas import tpu as pltpu

_NC = 2


def _moe_step(x_ref, rw_ref, gu_ref, dn_ref, out_ref, ew_ref, c_ref,
              *, num_experts):
    c = pl.program_id(0)
    e2 = pl.program_id(1)
    fi = pl.program_id(2)
    e = c * (num_experts // _NC) + e2
    first = jnp.logical_and(e2 == 0, fi == 0)

    @pl.when(first)
    def _routing():
        xv = x_ref[...]
        logits = jnp.dot(xv, rw_ref[...].T,
                         preferred_element_type=jnp.float32)
        m = jnp.max(logits, axis=1, keepdims=True)
        ex = jnp.exp(logits - m)
        s = ex / jnp.sum(ex, axis=1, keepdims=True)
        idx = jax.lax.broadcasted_iota(jnp.int32, s.shape, 1)
        m1 = jnp.max(s, axis=1, keepdims=True)
        i1 = jnp.min(jnp.where(s == m1, idx, num_experts), axis=1,
                     keepdims=True)
        s2 = jnp.where(idx == i1, -1.0, s)
        m2 = jnp.max(s2, axis=1, keepdims=True)
        i2 = jnp.min(jnp.where(s2 == m2, idx, num_experts), axis=1,
                     keepdims=True)
        c_ref[...] = (jnp.where(idx == i1, m1, 0.0)
                      + jnp.where(idx == i2, m2, 0.0))
        ew_ref[0] = jnp.concatenate([m1, m2], axis=1)

    xv = x_ref[...].astype(jnp.bfloat16)
    h = jnp.dot(xv, gu_ref[0].astype(jnp.bfloat16),
                preferred_element_type=jnp.float32)
    h_up = pltpu.roll(h, h.shape[1] - 1, 1)
    v = (h * jax.nn.sigmoid(h)) * h_up
    fc2 = h.shape[1]
    row = jax.lax.broadcasted_iota(jnp.int32, (fc2, fc2 // 2), 0)
    col = jax.lax.broadcasted_iota(jnp.int32, (fc2, fc2 // 2), 1)
    s_even = (row == 2 * col).astype(jnp.bfloat16)
    act = jnp.dot(v.astype(jnp.bfloat16), s_even,
                  preferred_element_type=jnp.float32)
    part = jnp.dot(act.astype(jnp.bfloat16),
                   dn_ref[0].astype(jnp.bfloat16),
                   preferred_element_type=jnp.float32)
    onehot = (jax.lax.broadcasted_iota(jnp.int32, (num_experts, 1), 0)
              == e).astype(jnp.float32)
    part = part * jnp.dot(c_ref[...], onehot,
                          preferred_element_type=jnp.float32)

    @pl.when(first)
    def _init():
        out_ref[0] = part

    @pl.when(jnp.logical_not(first))
    def _acc():
        out_ref[0] += part


def kernel(x, router_weight, gate_up_proj, down_proj):
    in_shape = x.shape
    xf = x.reshape(-1, x.shape[-1])
    T, H = xf.shape
    E, _, F2 = gate_up_proj.shape
    F = F2 // 2
    FC = 1024
    NF = F // FC
    EC = E // _NC

    out, ew = pl.pallas_call(
        functools.partial(_moe_step, num_experts=E),
        grid=(_NC, EC, NF),
        in_specs=[
            pl.BlockSpec((T, H), lambda c, e, fi: (0, 0)),
            pl.BlockSpec((E, H), lambda c, e, fi: (0, 0)),
            pl.BlockSpec((1, H, 2 * FC), lambda c, e, fi: (c * EC + e, 0, fi)),
            pl.BlockSpec((1, FC, H), lambda c, e, fi: (c * EC + e, fi, 0)),
        ],
        out_specs=[
            pl.BlockSpec((1, T, H), lambda c, e, fi: (c, 0, 0)),
            pl.BlockSpec((1, T, 2), lambda c, e, fi: (c, 0, 0)),
        ],
        out_shape=[
            jax.ShapeDtypeStruct((_NC, T, H), jnp.float32),
            jax.ShapeDtypeStruct((_NC, T, 2), jnp.float32),
        ],
        scratch_shapes=[pltpu.VMEM((T, E), jnp.float32)],
        compiler_params=pltpu.CompilerParams(
            dimension_semantics=("parallel", "arbitrary", "arbitrary"),
        ),
    )(xf, router_weight, gate_up_proj, down_proj)

    return (out[0] + out[1]).reshape(in_shape), ew[0]


# speedup vs baseline: 1.0124x; 1.0124x over previous
import functools

import jax
import jax.numpy as jnp
from jax.experimental import pallas as pl
from jax.experimental.pallas import tpu as pltpu


def _moe_step(x_ref, rw_ref, gu_ref, dn_ref, out_ref, ew_ref, c_ref,
              *, num_experts):
    e = pl.program_id(0)
    fi = pl.program_id(1)
    first = jnp.logical_and(e == 0, fi == 0)

    @pl.when(first)
    def _routing():
        xv = x_ref[...]
        logits = jnp.dot(xv, rw_ref[...].T,
                         preferred_element_type=jnp.float32)
        m = jnp.max(logits, axis=1, keepdims=True)
        ex = jnp.exp(logits - m)
        s = ex / jnp.sum(ex, axis=1, keepdims=True)
        idx = jax.lax.broadcasted_iota(jnp.int32, s.shape, 1)
        m1 = jnp.max(s, axis=1, keepdims=True)
        i1 = jnp.min(jnp.where(s == m1, idx, num_experts), axis=1,
                     keepdims=True)
        s2 = jnp.where(idx == i1, -1.0, s)
        m2 = jnp.max(s2, axis=1, keepdims=True)
        i2 = jnp.min(jnp.where(s2 == m2, idx, num_experts), axis=1,
                     keepdims=True)
        c_ref[...] = (jnp.where(idx == i1, m1, 0.0)
                      + jnp.where(idx == i2, m2, 0.0))
        ew_ref[...] = jnp.concatenate([m1, m2], axis=1)

    xv = x_ref[...]
    gu = gu_ref[0]
    h = jnp.dot(xv, gu, preferred_element_type=jnp.float32)
    h_up = pltpu.roll(h, h.shape[1] - 1, 1)
    v = (h * jax.nn.sigmoid(h)) * h_up
    fc2 = h.shape[1]
    row = jax.lax.broadcasted_iota(jnp.int32, (fc2, fc2 // 2), 0)
    col = jax.lax.broadcasted_iota(jnp.int32, (fc2, fc2 // 2), 1)
    s_even = (row == 2 * col).astype(jnp.float32)
    act = jnp.dot(v, s_even, preferred_element_type=jnp.float32)
    part = jnp.dot(act, dn_ref[0],
                   preferred_element_type=jnp.float32)
    onehot = (jax.lax.broadcasted_iota(jnp.int32, (num_experts, 1), 0)
              == e).astype(jnp.float32)
    part = part * jnp.dot(c_ref[...], onehot,
                          preferred_element_type=jnp.float32)

    @pl.when(first)
    def _init():
        out_ref[...] = part

    @pl.when(jnp.logical_not(first))
    def _acc():
        out_ref[...] += part


def kernel(x, router_weight, gate_up_proj, down_proj):
    in_shape = x.shape
    xf = x.reshape(-1, x.shape[-1])
    T, H = xf.shape
    E, _, F2 = gate_up_proj.shape
    F = F2 // 2
    FC = 1024
    NF = F // FC

    out, ew = pl.pallas_call(
        functools.partial(_moe_step, num_experts=E),
        grid=(E, NF),
        in_specs=[
            pl.BlockSpec((T, H), lambda e, fi: (0, 0)),
            pl.BlockSpec((E, H), lambda e, fi: (0, 0)),
            pl.BlockSpec((1, H, 2 * FC), lambda e, fi: (e, 0, fi)),
            pl.BlockSpec((1, FC, H), lambda e, fi: (e, fi, 0)),
        ],
        out_specs=[
            pl.BlockSpec((T, H), lambda e, fi: (0, 0)),
            pl.BlockSpec((T, 2), lambda e, fi: (0, 0)),
        ],
        out_shape=[
            jax.ShapeDtypeStruct((T, H), jnp.float32),
            jax.ShapeDtypeStruct((T, 2), jnp.float32),
        ],
        scratch_shapes=[pltpu.VMEM((T, E), jnp.float32)],
        compiler_params=pltpu.CompilerParams(
            dimension_semantics=("arbitrary", "arbitrary"),
        ),
    )(xf, router_weight, gate_up_proj, down_proj)

    return out.reshape(in_shape), ew
